# baseline (speedup 1.0000x reference)
import functools

import jax
import jax.numpy as jnp
from jax import lax
from jax.experimental import pallas as pl
from jax.experimental.pallas import tpu as pltpu
from jax.experimental.pallas import tpu_sc as plsc

N = 10000
E = 320000
D_FEAT = 128
LATENT = 16
HIDDEN = 32

NC = 2
NS = 16
NW = NC * NS
EPW = E // NW
CH = 1000
NCH = EPW // CH
NPT = N // NS

_mesh = plsc.VectorSubcoreMesh(core_axis_name="c", subcore_axis_name="s")
_sc_params = pltpu.CompilerParams(use_tc_tiling_on_sc=False)


_NB = 4
_LA = 2


@functools.partial(
    pl.kernel,
    out_type=(jax.ShapeDtypeStruct((E, LATENT), jnp.float32),
              jax.ShapeDtypeStruct((E, LATENT), jnp.float32)),
    mesh=_mesh,
    scratch_types=[pltpu.VMEM((NCH, CH), jnp.int32),
                   pltpu.VMEM((NCH, CH), jnp.int32),
                   [pltpu.VMEM((CH, LATENT), jnp.float32)] * _NB,
                   pltpu.SemaphoreType.DMA,
                   [pltpu.SemaphoreType.DMA] * _NB,
                   [pltpu.SemaphoreType.DMA] * _NB],
    compiler_params=_sc_params,
)
def _sc_gather(x_hbm, e128_hbm, snd_hbm, rcv_hbm, xs_hbm, xr_hbm,
               idxs_v, idxr_v, bufs, isem, gsems, wsems):
    del e128_hbm
    wid = lax.axis_index("s") * NC + lax.axis_index("c")
    base0 = pl.multiple_of(wid * EPW, 8)
    idescs = []
    for c in range(NCH):
        b = base0 + c * CH
        idescs.append(pltpu.async_copy(snd_hbm.at[pl.ds(b, CH)],
                                       idxs_v.at[c], isem))
        idescs.append(pltpu.async_copy(rcv_hbm.at[pl.ds(b, CH)],
                                       idxr_v.at[c], isem))
    for d in idescs:
        d.wait()

    nk = 2 * NCH
    g = [None] * _NB
    w = [None] * _NB
    for k in range(nk + _LA):
        if k < nk:
            j = k % _NB
            c = k // 2
            if w[j] is not None:
                w[j].wait()
            idx = idxs_v.at[c] if k % 2 == 0 else idxr_v.at[c]
            g[j] = pltpu.async_copy(x_hbm.at[idx], bufs[j], gsems[j])
        if k >= _LA:
            kp = k - _LA
            jp = kp % _NB
            cp = kp // 2
            b = base0 + cp * CH
            out = xs_hbm if kp % 2 == 0 else xr_hbm
            g[jp].wait()
            w[jp] = pltpu.async_copy(bufs[jp], out.at[pl.ds(b, CH)], wsems[jp])
    for j in range(_NB):
        if w[j] is not None:
            w[j].wait()


@functools.partial(
    pl.kernel,
    out_type=jax.ShapeDtypeStruct((NC, N, LATENT), jnp.float32),
    mesh=_mesh,
    scratch_types=[pltpu.VMEM((NCH, CH), jnp.int32),
                   [pltpu.VMEM((CH, LATENT), jnp.float32)] * 2,
                   pltpu.VMEM_SHARED((N, LATENT), jnp.float32),
                   pltpu.SemaphoreType.DMA,
                   [pltpu.SemaphoreType.DMA] * 2,
                   [pltpu.SemaphoreType.DMA] * 2],
    compiler_params=_sc_params,
)
def _sc_scatter(ne_hbm, rcv_hbm, zeros_hbm, out_hbm, idx_v, bufs, acc_sh,
                isem, lsems, asems):
    cid = lax.axis_index("c")
    sid = lax.axis_index("s")
    wid = sid * NC + cid
    base0 = pl.multiple_of(wid * EPW, 8)
    idescs = []
    for c in range(NCH):
        b = base0 + c * CH
        idescs.append(pltpu.async_copy(rcv_hbm.at[pl.ds(b, CH)],
                                       idx_v.at[c], isem))
    pltpu.sync_copy(zeros_hbm.at[pl.ds(sid * NPT, NPT)],
                    acc_sh.at[pl.ds(sid * NPT, NPT)])
    for d in idescs:
        d.wait()
    plsc.subcore_barrier()

    ld = [None] * 2
    ad = [None] * 2
    for c in range(NCH + 1):
        if c < NCH:
            j = c % 2
            b = base0 + c * CH
            if ad[j] is not None:
                ad[j].wait()
            ld[j] = pltpu.async_copy(ne_hbm.at[pl.ds(b, CH)], bufs[j],
                                     lsems[j])
        if c >= 1:
            jp = (c - 1) % 2
            ld[jp].wait()
            ad[jp] = pltpu.async_copy(bufs[jp], acc_sh.at[idx_v.at[c - 1]],
                                      asems[jp], add=True)
    for j in range(2):
        if ad[j] is not None:
            ad[j].wait()
    plsc.subcore_barrier()
    pltpu.sync_copy(acc_sh.at[pl.ds(sid * NPT, NPT)],
                    out_hbm.at[cid, pl.ds(sid * NPT, NPT)])


def _dot(a, b):
    return jnp.dot(a, b, preferred_element_type=jnp.float32)


def _embed_body(n_ref, w_ref, b_ref, x_ref):
    x_ref[...] = _dot(n_ref[...], w_ref[...]) + b_ref[...]


def _edge1_body(e_ref, xs_ref, xr_ref, w_ref, c1_ref, w2_ref, b2_ref, ne_ref):
    h = (_dot(e_ref[...], w_ref[0]) + _dot(xs_ref[...], w_ref[1])
         + _dot(xr_ref[...], w_ref[2]) + c1_ref[...])
    ne_ref[...] = _dot(jnp.maximum(h, 0.0), w2_ref[...]) + b2_ref[...]


def _edge2_body(e_ref, ne1_ref, xs_ref, xr_ref, w_ref, c1_ref, w2_ref, b2_ref,
                ne_ref):
    e2 = e_ref[...] + ne1_ref[...]
    h = (_dot(e2, w_ref[0]) + _dot(xs_ref[...], w_ref[1])
         + _dot(xr_ref[...], w_ref[2]) + c1_ref[...])
    ne_ref[...] = _dot(jnp.maximum(h, 0.0), w2_ref[...]) + b2_ref[...]


PK = 8
E8 = E // PK


def _ln(y, scale, bias):
    mu = jnp.mean(y, axis=-1, keepdims=True)
    var = jnp.mean((y - mu) * (y - mu), axis=-1, keepdims=True)
    return (y - mu) / jnp.sqrt(var + 1e-6) * scale + bias


def _node1_body(x_ref, part_ref, wx_ref, wr_ref, cn_ref, w2_ref, b2_ref,
                lns_ref, lnb_ref, xo_ref):
    x = x_ref[...]
    r = part_ref[0] + part_ref[1]
    h = jnp.maximum(_dot(x, wx_ref[...]) + _dot(r, wr_ref[...]) + cn_ref[...],
                    0.0)
    y = x + _dot(h, w2_ref[...]) + b2_ref[...]
    xo_ref[...] = _ln(y, lns_ref[...], lnb_ref[...])


def _node2_body(x_ref, part_ref, wx_ref, wr_ref, cn_ref, w2_ref, b2_ref,
                lns_ref, lnb_ref, dw_ref, db_ref, o_ref):
    x = x_ref[...]
    r = part_ref[0] + part_ref[1]
    h = jnp.maximum(_dot(x, wx_ref[...]) + _dot(r, wr_ref[...]) + cn_ref[...],
                    0.0)
    y = x + _dot(h, w2_ref[...]) + b2_ref[...]
    y = _ln(y, lns_ref[...], lnb_ref[...])
    o_ref[...] = _dot(y, dw_ref[...]) + db_ref[...]


def _full(shape):
    return pl.BlockSpec(shape, lambda *_: tuple(0 for _ in shape))


_BP = 1000

_tc_embed = pl.pallas_call(
    _embed_body,
    grid=(1,),
    in_specs=[_full((N, D_FEAT)), _full((D_FEAT, LATENT)), _full((1, LATENT))],
    out_specs=_full((N, LATENT)),
    out_shape=jax.ShapeDtypeStruct((N, LATENT), jnp.float32),
)

_edge_in_common = [_full((3, PK * LATENT, PK * HIDDEN)), _full((1, PK * HIDDEN)),
                   _full((PK * HIDDEN, PK * LATENT)), _full((1, PK * LATENT))]
_eblk = pl.BlockSpec((_BP, PK * LATENT), lambda i: (i, 0))

_tc_edge1 = pl.pallas_call(
    _edge1_body,
    grid=(E8 // _BP,),
    in_specs=[_eblk, _eblk, _eblk] + _edge_in_common,
    out_specs=_eblk,
    out_shape=jax.ShapeDtypeStruct((E8, PK * LATENT), jnp.float32),
)

_tc_edge2 = pl.pallas_call(
    _edge2_body,
    grid=(E8 // _BP,),
    in_specs=[_eblk, _eblk, _eblk, _eblk] + _edge_in_common,
    out_specs=_eblk,
    out_shape=jax.ShapeDtypeStruct((E8, PK * LATENT), jnp.float32),
)

_node_in_common = [
    _full((N, LATENT)), _full((NC, N, LATENT)),
    _full((LATENT, HIDDEN)), _full((LATENT, HIDDEN)), _full((1, HIDDEN)),
    _full((HIDDEN, LATENT)), _full((1, LATENT)),
    _full((1, LATENT)), _full((1, LATENT)),
]

_tc_node1 = pl.pallas_call(
    _node1_body,
    grid=(1,),
    in_specs=list(_node_in_common),
    out_specs=_full((N, LATENT)),
    out_shape=jax.ShapeDtypeStruct((N, LATENT), jnp.float32),
)

_tc_node2 = pl.pallas_call(
    _node2_body,
    grid=(1,),
    in_specs=list(_node_in_common) + [_full((LATENT, D_FEAT)),
                                      _full((1, D_FEAT))],
    out_specs=_full((N, D_FEAT)),
    out_shape=jax.ShapeDtypeStruct((N, D_FEAT), jnp.float32),
)


def kernel(nodes, edge_index, edges, globals_, params):
    senders = edge_index[0]
    receivers = edge_index[1]
    p = params
    x = _tc_embed(nodes, p["embed_W"], p["embed_b"].reshape(1, -1))
    zeros = jnp.zeros((N, LATENT), jnp.float32)
    ne1 = None
    out = None
    eye8 = jnp.eye(PK, dtype=jnp.float32)
    edges128 = edges.reshape(E8, PK * LATENT)
    for step in range(2):
        sp = p["steps"][step]
        gs = (globals_ * (2.0 ** step)).reshape(1, -1)
        (W1, b1), (W2, b2) = sp["edge"]
        wstack = jnp.stack([jnp.kron(eye8, W1[0:LATENT]),
                            jnp.kron(eye8, W1[LATENT:2 * LATENT]),
                            jnp.kron(eye8, W1[2 * LATENT:3 * LATENT])])
        c1 = jnp.tile(gs @ W1[3 * LATENT:] + b1, (1, PK))
        w2bd = jnp.kron(eye8, W2)
        b2t = jnp.tile(b2.reshape(1, -1), (1, PK))
        (Wn1, bn1), (Wn2, bn2) = sp["node"]
        cn = gs @ Wn1[2 * LATENT:] + bn1

        xs, xr = _sc_gather(x, edges128, senders, receivers)
        xs128 = xs.reshape(E8, PK * LATENT)
        xr128 = xr.reshape(E8, PK * LATENT)
        if step == 0:
            ne1 = _tc_edge1(edges128, xs128, xr128, wstack, c1, w2bd, b2t)
            ne = ne1
        else:
            ne = _tc_edge2(edges128, ne1, xs128, xr128, wstack, c1, w2bd, b2t)
        part = _sc_scatter(ne.reshape(E, LATENT), receivers, zeros)
        nargs = (x, part, Wn1[0:LATENT], Wn1[LATENT:2 * LATENT], cn,
                 Wn2, bn2.reshape(1, -1), sp["ln_scale"].reshape(1, -1),
                 sp["ln_bias"].reshape(1, -1))
        if step == 0:
            x = _tc_node1(*nargs)
        else:
            out = _tc_node2(*nargs, p["dec_W"], p["dec_b"].reshape(1, -1))
    return out


# speedup vs baseline: 1.2397x; 1.2397x over previous
import functools

import jax
import jax.numpy as jnp
from jax import lax
from jax.experimental import pallas as pl
from jax.experimental.pallas import tpu as pltpu
from jax.experimental.pallas import tpu_sc as plsc

N = 10000
E = 320000
D_FEAT = 128
LATENT = 16
HIDDEN = 32

NC = 2
NS = 16
NW = NC * NS
EPW = E // NW
CH = 2000
NCH = EPW // CH
NPT = N // NS

_mesh = plsc.VectorSubcoreMesh(core_axis_name="c", subcore_axis_name="s")
_sc_params = pltpu.CompilerParams(use_tc_tiling_on_sc=False)


_NB = 3
_LA = 1


@functools.partial(
    pl.kernel,
    out_type=(jax.ShapeDtypeStruct((E, LATENT), jnp.float32),
              jax.ShapeDtypeStruct((E, LATENT), jnp.float32)),
    mesh=_mesh,
    scratch_types=[pltpu.VMEM((NCH, CH), jnp.int32),
                   pltpu.VMEM((NCH, CH), jnp.int32),
                   [pltpu.VMEM((CH, LATENT), jnp.float32)] * _NB,
                   pltpu.SemaphoreType.DMA,
                   [pltpu.SemaphoreType.DMA] * _NB,
                   [pltpu.SemaphoreType.DMA] * _NB],
    compiler_params=_sc_params,
)
def _sc_gather(x_hbm, snd_hbm, rcv_hbm, xs_hbm, xr_hbm,
               idxs_v, idxr_v, bufs, isem, gsems, wsems):
    wid = lax.axis_index("s") * NC + lax.axis_index("c")
    base0 = pl.multiple_of(wid * EPW, 8)
    idescs = []
    for c in range(NCH):
        b = base0 + c * CH
        idescs.append(pltpu.async_copy(snd_hbm.at[pl.ds(b, CH)],
                                       idxs_v.at[c], isem))
        idescs.append(pltpu.async_copy(rcv_hbm.at[pl.ds(b, CH)],
                                       idxr_v.at[c], isem))
    iwaited = 0

    nk = 2 * NCH
    g = [None] * _NB
    w = [None] * _NB
    for k in range(nk + _LA):
        if k < nk:
            j = k % _NB
            c = k // 2
            if w[j] is not None:
                w[j].wait()
            while iwaited <= k:
                idescs[iwaited].wait()
                iwaited += 1
            idx = idxs_v.at[c] if k % 2 == 0 else idxr_v.at[c]
            g[j] = pltpu.async_copy(x_hbm.at[idx], bufs[j], gsems[j])
        if k >= _LA:
            kp = k - _LA
            jp = kp % _NB
            cp = kp // 2
            b = base0 + cp * CH
            out = xs_hbm if kp % 2 == 0 else xr_hbm
            g[jp].wait()
            w[jp] = pltpu.async_copy(bufs[jp], out.at[pl.ds(b, CH)], wsems[jp])
    for j in range(_NB):
        if w[j] is not None:
            w[j].wait()


@functools.partial(
    pl.kernel,
    out_type=jax.ShapeDtypeStruct((NC, N, LATENT), jnp.float32),
    mesh=_mesh,
    scratch_types=[pltpu.VMEM((NCH, CH), jnp.int32),
                   [pltpu.VMEM((CH, LATENT), jnp.float32)] * 2,
                   pltpu.VMEM_SHARED((N, LATENT), jnp.float32),
                   pltpu.SemaphoreType.DMA,
                   [pltpu.SemaphoreType.DMA] * 2,
                   [pltpu.SemaphoreType.DMA] * 2],
    compiler_params=_sc_params,
)
def _sc_scatter(ne_hbm, rcv_hbm, zeros_hbm, out_hbm, idx_v, bufs, acc_sh,
                isem, lsems, asems):
    cid = lax.axis_index("c")
    sid = lax.axis_index("s")
    wid = sid * NC + cid
    base0 = pl.multiple_of(wid * EPW, 8)
    idescs = []
    for c in range(NCH):
        b = base0 + c * CH
        idescs.append(pltpu.async_copy(rcv_hbm.at[pl.ds(b, CH)],
                                       idx_v.at[c], isem))
    pltpu.sync_copy(zeros_hbm.at[pl.ds(sid * NPT, NPT)],
                    acc_sh.at[pl.ds(sid * NPT, NPT)])
    for d in idescs:
        d.wait()
    plsc.subcore_barrier()

    ld = [None] * 2
    ad = [None] * 2
    for c in range(NCH + 1):
        if c < NCH:
            j = c % 2
            b = base0 + c * CH
            if ad[j] is not None:
                ad[j].wait()
            ld[j] = pltpu.async_copy(ne_hbm.at[pl.ds(b, CH)], bufs[j],
                                     lsems[j])
        if c >= 1:
            jp = (c - 1) % 2
            ld[jp].wait()
            ad[jp] = pltpu.async_copy(bufs[jp], acc_sh.at[idx_v.at[c - 1]],
                                      asems[jp], add=True)
    for j in range(2):
        if ad[j] is not None:
            ad[j].wait()
    plsc.subcore_barrier()
    pltpu.sync_copy(acc_sh.at[pl.ds(sid * NPT, NPT)],
                    out_hbm.at[cid, pl.ds(sid * NPT, NPT)])


def _dot(a, b):
    return jnp.dot(a, b, preferred_element_type=jnp.float32)


def _embed_body(n_ref, w_ref, b_ref, x_ref):
    x_ref[...] = _dot(n_ref[...], w_ref[...]) + b_ref[...]


def _edge1_body(e_ref, xs_ref, xr_ref, w_ref, c1_ref, w2_ref, b2_ref, ne_ref):
    h = (_dot(e_ref[...], w_ref[0]) + _dot(xs_ref[...], w_ref[1])
         + _dot(xr_ref[...], w_ref[2]) + c1_ref[...])
    ne_ref[...] = _dot(jnp.maximum(h, 0.0), w2_ref[...]) + b2_ref[...]


def _edge2_body(e_ref, ne1_ref, xs_ref, xr_ref, w_ref, c1_ref, w2_ref, b2_ref,
                ne_ref):
    e2 = e_ref[...] + ne1_ref[...]
    h = (_dot(e2, w_ref[0]) + _dot(xs_ref[...], w_ref[1])
         + _dot(xr_ref[...], w_ref[2]) + c1_ref[...])
    ne_ref[...] = _dot(jnp.maximum(h, 0.0), w2_ref[...]) + b2_ref[...]


PK = 8
E8 = E // PK


def _ln(y, scale, bias):
    mu = jnp.mean(y, axis=-1, keepdims=True)
    var = jnp.mean((y - mu) * (y - mu), axis=-1, keepdims=True)
    return (y - mu) / jnp.sqrt(var + 1e-6) * scale + bias


def _node1_body(x_ref, part_ref, wx_ref, wr_ref, cn_ref, w2_ref, b2_ref,
                lns_ref, lnb_ref, xo_ref):
    x = x_ref[...]
    r = part_ref[0] + part_ref[1]
    h = jnp.maximum(_dot(x, wx_ref[...]) + _dot(r, wr_ref[...]) + cn_ref[...],
                    0.0)
    y = x + _dot(h, w2_ref[...]) + b2_ref[...]
    xo_ref[...] = _ln(y, lns_ref[...], lnb_ref[...])


def _node2_body(x_ref, part_ref, wx_ref, wr_ref, cn_ref, w2_ref, b2_ref,
                lns_ref, lnb_ref, dw_ref, db_ref, o_ref):
    x = x_ref[...]
    r = part_ref[0] + part_ref[1]
    h = jnp.maximum(_dot(x, wx_ref[...]) + _dot(r, wr_ref[...]) + cn_ref[...],
                    0.0)
    y = x + _dot(h, w2_ref[...]) + b2_ref[...]
    y = _ln(y, lns_ref[...], lnb_ref[...])
    o_ref[...] = _dot(y, dw_ref[...]) + db_ref[...]


def _full(shape):
    return pl.BlockSpec(shape, lambda *_: tuple(0 for _ in shape))


_BP = 1000

_tc_embed = pl.pallas_call(
    _embed_body,
    grid=(1,),
    in_specs=[_full((N, D_FEAT)), _full((D_FEAT, LATENT)), _full((1, LATENT))],
    out_specs=_full((N, LATENT)),
    out_shape=jax.ShapeDtypeStruct((N, LATENT), jnp.float32),
)

_edge_in_common = [_full((3, PK * LATENT, PK * HIDDEN)), _full((1, PK * HIDDEN)),
                   _full((PK * HIDDEN, PK * LATENT)), _full((1, PK * LATENT))]
_eblk = pl.BlockSpec((_BP, PK * LATENT), lambda i: (i, 0))

_tc_edge1 = pl.pallas_call(
    _edge1_body,
    grid=(E8 // _BP,),
    in_specs=[_eblk, _eblk, _eblk] + _edge_in_common,
    out_specs=_eblk,
    out_shape=jax.ShapeDtypeStruct((E8, PK * LATENT), jnp.float32),
)

_tc_edge2 = pl.pallas_call(
    _edge2_body,
    grid=(E8 // _BP,),
    in_specs=[_eblk, _eblk, _eblk, _eblk] + _edge_in_common,
    out_specs=_eblk,
    out_shape=jax.ShapeDtypeStruct((E8, PK * LATENT), jnp.float32),
)

_node_in_common = [
    _full((N, LATENT)), _full((NC, N, LATENT)),
    _full((LATENT, HIDDEN)), _full((LATENT, HIDDEN)), _full((1, HIDDEN)),
    _full((HIDDEN, LATENT)), _full((1, LATENT)),
    _full((1, LATENT)), _full((1, LATENT)),
]

_tc_node1 = pl.pallas_call(
    _node1_body,
    grid=(1,),
    in_specs=list(_node_in_common),
    out_specs=_full((N, LATENT)),
    out_shape=jax.ShapeDtypeStruct((N, LATENT), jnp.float32),
)

_tc_node2 = pl.pallas_call(
    _node2_body,
    grid=(1,),
    in_specs=list(_node_in_common) + [_full((LATENT, D_FEAT)),
                                      _full((1, D_FEAT))],
    out_specs=_full((N, D_FEAT)),
    out_shape=jax.ShapeDtypeStruct((N, D_FEAT), jnp.float32),
)


def kernel(nodes, edge_index, edges, globals_, params):
    senders = edge_index[0]
    receivers = edge_index[1]
    p = params
    x = _tc_embed(nodes, p["embed_W"], p["embed_b"].reshape(1, -1))
    zeros = jnp.zeros((N, LATENT), jnp.float32)
    ne1 = None
    out = None
    eye8 = jnp.eye(PK, dtype=jnp.float32)
    edges128 = edges.reshape(E8, PK * LATENT)
    for step in range(2):
        sp = p["steps"][step]
        gs = (globals_ * (2.0 ** step)).reshape(1, -1)
        (W1, b1), (W2, b2) = sp["edge"]
        wstack = jnp.stack([jnp.kron(eye8, W1[0:LATENT]),
                            jnp.kron(eye8, W1[LATENT:2 * LATENT]),
                            jnp.kron(eye8, W1[2 * LATENT:3 * LATENT])])
        c1 = jnp.tile(gs @ W1[3 * LATENT:] + b1, (1, PK))
        w2bd = jnp.kron(eye8, W2)
        b2t = jnp.tile(b2.reshape(1, -1), (1, PK))
        (Wn1, bn1), (Wn2, bn2) = sp["node"]
        cn = gs @ Wn1[2 * LATENT:] + bn1

        xs, xr = _sc_gather(x, senders, receivers)
        xs128 = xs.reshape(E8, PK * LATENT)
        xr128 = xr.reshape(E8, PK * LATENT)
        if step == 0:
            ne1 = _tc_edge1(edges128, xs128, xr128, wstack, c1, w2bd, b2t)
            ne = ne1
        else:
            ne = _tc_edge2(edges128, ne1, xs128, xr128, wstack, c1, w2bd, b2t)
        part = _sc_scatter(ne.reshape(E, LATENT), receivers, zeros)
        nargs = (x, part, Wn1[0:LATENT], Wn1[LATENT:2 * LATENT], cn,
                 Wn2, bn2.reshape(1, -1), sp["ln_scale"].reshape(1, -1),
                 sp["ln_bias"].reshape(1, -1))
        if step == 0:
            x = _tc_node1(*nargs)
        else:
            out = _tc_node2(*nargs, p["dec_W"], p["dec_b"].reshape(1, -1))
    return out
